# baseline (speedup 1.0000x reference)
import functools

import jax
import jax.numpy as jnp
from jax import lax
from jax.experimental import pallas as pl
from jax.experimental.pallas import tpu as pltpu
from jax.experimental.pallas import tpu_sc as plsc

NC = 2
NS = 16
CHUNK = 128
KB = 4


def _cdiv(a, b):
    return (a + b - 1) // b


def kernel(x, edge_index, W, b, gamma, beta):
    n, d = x.shape
    e = edge_index.shape[1]
    dh = d // NC
    nw = NC * NS
    cpw = _cdiv(e, CHUNK * NS)
    cpw = _cdiv(cpw, 2 * KB) * 2 * KB
    e_pad = cpw * NS * CHUNK
    n_acc = _cdiv(n + 1, NS * CHUNK) * NS * CHUNK
    rpt = n_acc // NS
    wch = rpt // CHUNK
    n_grp = cpw // KB
    tab_rows = n + 8

    row = edge_index[0]
    col = edge_index[1]
    pad = e_pad - e
    if pad:
        row = jnp.concatenate([row, jnp.full((pad,), n, jnp.int32)])
        col = jnp.concatenate([col, jnp.full((pad,), n, jnp.int32)])
    row2 = row.reshape(NS, cpw, CHUNK)
    col2 = col.reshape(NS, cpw, CHUNK)
    row3 = jnp.concatenate([row2, row2 + tab_rows], axis=0)
    col3 = jnp.concatenate([col2, col2], axis=0)

    zrows = jnp.zeros((CHUNK, dh), jnp.float32)
    zpad8 = jnp.zeros((8, dh), jnp.float32)
    ones_tab = jnp.concatenate(
        [jnp.ones((n, dh), jnp.float32), zpad8] * NC)

    mesh = plsc.VectorSubcoreMesh(core_axis_name="c", subcore_axis_name="s")

    @functools.partial(
        pl.kernel,
        out_type=jax.ShapeDtypeStruct((NC, n_acc, dh), jnp.float32),
        mesh=mesh,
        scratch_types=[
            pltpu.VMEM((cpw, CHUNK), jnp.int32),
            pltpu.VMEM((cpw, CHUNK), jnp.int32),
        ] + [pltpu.VMEM((CHUNK, dh), jnp.float32)] * KB + [
            pltpu.VMEM_SHARED((n_acc, dh), jnp.float32),
        ] + [pltpu.SemaphoreType.DMA] * (2 * KB),
        compiler_params=pltpu.CompilerParams(use_tc_tiling_on_sc=False),
    )
    def agg_kernel(tab_hbm, gidx_hbm, sidx_hbm, z_hbm, out_hbm,
                   ridx2, cidx2, *rest):
        bufs = rest[:KB]
        acc = rest[KB]
        gsem = rest[KB + 1:KB + 1 + KB]
        ssem = rest[KB + 1 + KB:]
        cid = lax.axis_index("c")
        sid = lax.axis_index("s")
        wid = cid * NS + sid
        r0 = sid * rpt
        pltpu.sync_copy(gidx_hbm.at[wid], ridx2)
        pltpu.sync_copy(sidx_hbm.at[wid], cidx2)
        pltpu.sync_copy(z_hbm, bufs[0])
        for k in range(wch):
            pltpu.sync_copy(bufs[0], acc.at[pl.ds(r0 + k * CHUNK, CHUNK)])
        plsc.subcore_barrier()

        def body(g, carry):
            gd = []
            for k in range(KB):
                c = g * KB + k
                gd.append(pltpu.async_copy(
                    tab_hbm.at[ridx2.at[c]], bufs[k], gsem[k]))
            sd = []
            for k in range(KB):
                c = g * KB + k
                gd[k].wait()
                sd.append(pltpu.async_copy(
                    bufs[k], acc.at[cidx2.at[c]], ssem[k], add=True))
            for dsc in sd:
                dsc.wait()
            return carry

        lax.fori_loop(0, n_grp, body, 0)
        plsc.subcore_barrier()
        for k in range(wch):
            pltpu.sync_copy(acc.at[pl.ds(r0 + k * CHUNK, CHUNK)], bufs[0])
            pltpu.sync_copy(bufs[0], out_hbm.at[cid, pl.ds(r0 + k * CHUNK, CHUNK)])

    degp = agg_kernel(ones_tab, col3, col3, zrows)

    BM = 2000

    def mm_body(x_ref, w_ref, o_ref):
        o_ref[...] = jnp.dot(x_ref[...], w_ref[...],
                             preferred_element_type=jnp.float32)

    xw = pl.pallas_call(
        mm_body,
        grid=(n // BM,),
        in_specs=[pl.BlockSpec((BM, d), lambda i: (i, 0)),
                  pl.BlockSpec((d, d), lambda i: (0, 0))],
        out_specs=pl.BlockSpec((BM, d), lambda i: (i, 0)),
        out_shape=jax.ShapeDtypeStruct((n, d), jnp.float32),
    )(x, W)

    dp0 = degp[0, :n, 0:1]

    def s_body(xw_ref, d0_ref, y_ref, dv_ref):
        deg = d0_ref[...]
        dinv = jnp.where(deg > 0.0,
                         lax.rsqrt(jnp.maximum(deg, 1e-12)), 0.0)
        y_ref[...] = xw_ref[...] * dinv
        dv_ref[...] = dinv

    y, dinv = pl.pallas_call(
        s_body,
        grid=(n // BM,),
        in_specs=[pl.BlockSpec((BM, d), lambda i: (i, 0)),
                  pl.BlockSpec((BM, 1), lambda i: (i, 0))],
        out_specs=[pl.BlockSpec((BM, d), lambda i: (i, 0)),
                   pl.BlockSpec((BM, 1), lambda i: (i, 0))],
        out_shape=[jax.ShapeDtypeStruct((n, d), jnp.float32),
                   jax.ShapeDtypeStruct((n, 1), jnp.float32)],
    )(xw, dp0)

    y_tab = jnp.concatenate([y[:, :dh], zpad8, y[:, dh:], zpad8])

    aggp = agg_kernel(y_tab, row3, col3, zrows)

    a0 = aggp[0, :n]
    a1 = aggp[1, :n]
    b2 = b.reshape(1, d)
    g2 = gamma.reshape(1, d)
    be2 = beta.reshape(1, d)

    def f1_body(a0_ref, a1_ref, dv_ref, b_ref, q_ref, st_ref):
        i = pl.program_id(0)
        dv = dv_ref[...]
        bb = b_ref[...]
        q0 = a0_ref[...] * dv + bb[:, :dh]
        q1 = a1_ref[...] * dv + bb[:, dh:]
        q = jnp.concatenate([q0, q1], axis=1)
        q_ref[...] = q

        @pl.when(i == 0)
        def _():
            st_ref[...] = jnp.zeros_like(st_ref)

        st_ref[0:1, :] += jnp.sum(q, axis=0, keepdims=True)
        st_ref[1:2, :] += jnp.sum(q * q, axis=0, keepdims=True)

    q, stats = pl.pallas_call(
        f1_body,
        grid=(n // BM,),
        in_specs=[pl.BlockSpec((BM, dh), lambda i: (i, 0)),
                  pl.BlockSpec((BM, dh), lambda i: (i, 0)),
                  pl.BlockSpec((BM, 1), lambda i: (i, 0)),
                  pl.BlockSpec((1, d), lambda i: (0, 0))],
        out_specs=[pl.BlockSpec((BM, d), lambda i: (i, 0)),
                   pl.BlockSpec((2, d), lambda i: (0, 0))],
        out_shape=[jax.ShapeDtypeStruct((n, d), jnp.float32),
                   jax.ShapeDtypeStruct((2, d), jnp.float32)],
    )(a0, a1, dinv, b2)

    def f2_body(q_ref, st_ref, g_ref, be_ref, x_ref, o_ref):
        mean = st_ref[0:1, :] * (1.0 / n)
        var = st_ref[1:2, :] * (1.0 / n) - mean * mean
        hh = (g_ref[...] * (q_ref[...] - mean) * lax.rsqrt(var + 1e-5)
              + be_ref[...])
        o_ref[...] = jnp.maximum(hh, 0.0) + x_ref[...]

    h = pl.pallas_call(
        f2_body,
        grid=(n // BM,),
        in_specs=[pl.BlockSpec((BM, d), lambda i: (i, 0)),
                  pl.BlockSpec((2, d), lambda i: (0, 0)),
                  pl.BlockSpec((1, d), lambda i: (0, 0)),
                  pl.BlockSpec((1, d), lambda i: (0, 0)),
                  pl.BlockSpec((BM, d), lambda i: (i, 0))],
        out_specs=pl.BlockSpec((BM, d), lambda i: (i, 0)),
        out_shape=jax.ShapeDtypeStruct((n, d), jnp.float32),
    )(q, stats, g2, be2, x)

    return h


# speedup vs baseline: 1.9831x; 1.9831x over previous
import functools

import jax
import jax.numpy as jnp
from jax import lax
from jax.experimental import pallas as pl
from jax.experimental.pallas import tpu as pltpu
from jax.experimental.pallas import tpu_sc as plsc

NC = 2
NS = 16
CHB = 176
CHD = 256
DEG_W = 16
ROWCH = 128
KB = 4
KD = 4


def _cdiv(a, b):
    return (a + b - 1) // b


def kernel(x, edge_index, W, b, gamma, beta):
    n, d = x.shape
    e = edge_index.shape[1]
    dh = d // NC
    cpb = _cdiv(e, CHB * NS)
    cpb = _cdiv(cpb, KB) * KB
    e_pad = cpb * CHB * NS
    n_acc = _cdiv(n + 1, NS * ROWCH) * NS * ROWCH
    rpt = n_acc // NS
    wch = rpt // ROWCH
    n_grp_b = cpb // KB
    nw = NC * NS
    cpd = _cdiv(e, CHD * nw)
    cpd = _cdiv(cpd, KD) * KD
    e_pad_d = cpd * CHD * nw
    n_grp_d = cpd // KD
    tab_rows = n_acc

    row = edge_index[0]
    col = edge_index[1]
    padv = jnp.full((e_pad - e,), n, jnp.int32)
    padd = jnp.full((e_pad_d - e,), n, jnp.int32)
    rowb = jnp.concatenate([row, padv]).reshape(NS, cpb, CHB)
    colb = jnp.concatenate([col, padv]).reshape(NS, cpb, CHB)
    row3 = jnp.concatenate([rowb, rowb + tab_rows], axis=0)
    cold = jnp.concatenate([col, padd]).reshape(nw, cpd, CHD)

    zrows = jnp.zeros((CHB, dh), jnp.float32)
    onesw = jnp.ones((CHD, DEG_W), jnp.float32)
    zerosw = jnp.zeros((CHD, DEG_W), jnp.float32)

    mesh = plsc.VectorSubcoreMesh(core_axis_name="c", subcore_axis_name="s")

    @functools.partial(
        pl.kernel,
        out_type=jax.ShapeDtypeStruct((NC, n_acc, dh), jnp.float32),
        mesh=mesh,
        scratch_types=[
            pltpu.VMEM((cpb, CHB), jnp.int32),
            pltpu.VMEM((cpb, CHB), jnp.int32),
        ] + [pltpu.VMEM((CHB, dh), jnp.float32)] * KB + [
            pltpu.VMEM_SHARED((n_acc, dh), jnp.float32),
        ] + [pltpu.SemaphoreType.DMA] * (2 * KB),
        compiler_params=pltpu.CompilerParams(use_tc_tiling_on_sc=False),
    )
    def agg_kernel(tab_hbm, gidx_hbm, sidx_hbm, z_hbm, out_hbm,
                   ridx2, cidx2, *rest):
        bufs = rest[:KB]
        acc = rest[KB]
        gsem = rest[KB + 1:KB + 1 + KB]
        ssem = rest[KB + 1 + KB:]
        cid = lax.axis_index("c")
        sid = lax.axis_index("s")
        wid = cid * NS + sid
        r0 = sid * rpt
        pltpu.sync_copy(gidx_hbm.at[wid], ridx2)
        pltpu.sync_copy(sidx_hbm.at[sid], cidx2)
        pltpu.sync_copy(z_hbm, bufs[0])
        for k in range(wch):
            pltpu.sync_copy(bufs[0].at[pl.ds(0, ROWCH)],
                            acc.at[pl.ds(r0 + k * ROWCH, ROWCH)])
        plsc.subcore_barrier()

        def body(g, carry):
            gd = []
            for k in range(KB):
                c = g * KB + k
                gd.append(pltpu.async_copy(
                    tab_hbm.at[ridx2.at[c]], bufs[k], gsem[k]))
            sd = []
            for k in range(KB):
                c = g * KB + k
                gd[k].wait()
                sd.append(pltpu.async_copy(
                    bufs[k], acc.at[cidx2.at[c]], ssem[k], add=True))
            for dsc in sd:
                dsc.wait()
            return carry

        lax.fori_loop(0, n_grp_b, body, 0)
        plsc.subcore_barrier()
        for k in range(wch):
            pltpu.sync_copy(acc.at[pl.ds(r0 + k * ROWCH, ROWCH)],
                            bufs[0].at[pl.ds(0, ROWCH)])
            pltpu.sync_copy(bufs[0].at[pl.ds(0, ROWCH)],
                            out_hbm.at[cid, pl.ds(r0 + k * ROWCH, ROWCH)])


    @functools.partial(
        pl.kernel,
        out_type=jax.ShapeDtypeStruct((NC, n_acc, DEG_W), jnp.float32),
        mesh=mesh,
        scratch_types=[
            pltpu.VMEM((cpd, CHD), jnp.int32),
            pltpu.VMEM((CHD, DEG_W), jnp.float32),
            pltpu.VMEM((CHD, DEG_W), jnp.float32),
            pltpu.VMEM_SHARED((n_acc, DEG_W), jnp.float32),
        ] + [pltpu.SemaphoreType.DMA] * KD,
        compiler_params=pltpu.CompilerParams(use_tc_tiling_on_sc=False),
    )
    def deg_kernel(cold_hbm, ones_hbm, zeros_hbm, out_hbm,
                   cidx2, vone, vzero, acc, *sems):
        cid = lax.axis_index("c")
        sid = lax.axis_index("s")
        wid = cid * NS + sid
        r0 = sid * rpt
        pltpu.sync_copy(cold_hbm.at[wid], cidx2)
        pltpu.sync_copy(zeros_hbm, vzero)
        pltpu.sync_copy(ones_hbm, vone)
        for k in range(wch):
            pltpu.sync_copy(vzero.at[pl.ds(0, ROWCH)],
                            acc.at[pl.ds(r0 + k * ROWCH, ROWCH)])
        plsc.subcore_barrier()

        def body(g, carry):
            descs = []
            for k in range(KD):
                c = g * KD + k
                descs.append(pltpu.async_copy(
                    vone, acc.at[cidx2.at[c]], sems[k], add=True))
            for dsc in descs:
                dsc.wait()
            return carry

        lax.fori_loop(0, n_grp_d, body, 0)
        plsc.subcore_barrier()
        for k in range(wch):
            pltpu.sync_copy(acc.at[pl.ds(r0 + k * ROWCH, ROWCH)],
                            vzero.at[pl.ds(0, ROWCH)])
            pltpu.sync_copy(vzero.at[pl.ds(0, ROWCH)],
                            out_hbm.at[cid, pl.ds(r0 + k * ROWCH, ROWCH)])

    degp = deg_kernel(cold, onesw, zerosw)

    BM = 2000

    dp0 = degp[0, :n, 0:1]
    dp1 = degp[1, :n, 0:1]
    BM2 = n_acc // 5

    def s_body(x_ref, w_ref, d0_ref, d1_ref, yt_ref, dv_ref):
        i = pl.program_id(0)
        ridx = lax.broadcasted_iota(jnp.int32, (BM2, 1), 0) + i * BM2
        valid = ridx < n
        deg = d0_ref[...] + d1_ref[...]
        dinv = jnp.where(jnp.logical_and(valid, deg > 0.0),
                         lax.rsqrt(jnp.maximum(deg, 1e-12)), 0.0)
        xw = jnp.dot(x_ref[...], w_ref[...], preferred_element_type=jnp.float32)
        y = jnp.where(valid, xw * dinv, 0.0)
        yt_ref[0] = y[:, :dh]
        yt_ref[1] = y[:, dh:]
        dv_ref[...] = dinv

    yt, dinv = pl.pallas_call(
        s_body,
        grid=(5,),
        in_specs=[pl.BlockSpec((BM2, d), lambda i: (i, 0)),
                  pl.BlockSpec((d, d), lambda i: (0, 0)),
                  pl.BlockSpec((BM2, 1), lambda i: (i, 0)),
                  pl.BlockSpec((BM2, 1), lambda i: (i, 0))],
        out_specs=[pl.BlockSpec((NC, BM2, dh), lambda i: (0, i, 0)),
                   pl.BlockSpec((BM2, 1), lambda i: (i, 0))],
        out_shape=[jax.ShapeDtypeStruct((NC, n_acc, dh), jnp.float32),
                   jax.ShapeDtypeStruct((n_acc, 1), jnp.float32)],
    )(x, W, dp0, dp1)

    y_tab = yt.reshape(NC * n_acc, dh)
    dinv = dinv[:n]

    aggp = agg_kernel(y_tab, row3, colb, zrows)

    a0 = aggp[0, :n]
    a1 = aggp[1, :n]
    b2 = b.reshape(1, d)
    g2 = gamma.reshape(1, d)
    be2 = beta.reshape(1, d)

    def f1_body(a0_ref, a1_ref, dv_ref, b_ref, q_ref, st_ref):
        i = pl.program_id(0)
        dv = dv_ref[...]
        bb = b_ref[...]
        q = jnp.concatenate([a0_ref[...] * dv + bb[:, :dh],
                             a1_ref[...] * dv + bb[:, dh:]], axis=1)
        q_ref[...] = q

        @pl.when(i == 0)
        def _():
            st_ref[...] = jnp.zeros_like(st_ref)

        st_ref[0:1, :] += jnp.sum(q, axis=0, keepdims=True)
        st_ref[1:2, :] += jnp.sum(q * q, axis=0, keepdims=True)

    q, stats = pl.pallas_call(
        f1_body,
        grid=(n // BM,),
        in_specs=[pl.BlockSpec((BM, dh), lambda i: (i, 0)),
                  pl.BlockSpec((BM, dh), lambda i: (i, 0)),
                  pl.BlockSpec((BM, 1), lambda i: (i, 0)),
                  pl.BlockSpec((1, d), lambda i: (0, 0))],
        out_specs=[pl.BlockSpec((BM, d), lambda i: (i, 0)),
                   pl.BlockSpec((2, d), lambda i: (0, 0))],
        out_shape=[jax.ShapeDtypeStruct((n, d), jnp.float32),
                   jax.ShapeDtypeStruct((2, d), jnp.float32)],
    )(a0, a1, dinv, b2)

    def f2_body(q_ref, st_ref, g_ref, be_ref, x_ref, o_ref):
        mean = st_ref[0:1, :] * (1.0 / n)
        var = st_ref[1:2, :] * (1.0 / n) - mean * mean
        hh = (g_ref[...] * (q_ref[...] - mean) * lax.rsqrt(var + 1e-5)
              + be_ref[...])
        o_ref[...] = jnp.maximum(hh, 0.0) + x_ref[...]

    h = pl.pallas_call(
        f2_body,
        grid=(n // BM,),
        in_specs=[pl.BlockSpec((BM, d), lambda i: (i, 0)),
                  pl.BlockSpec((2, d), lambda i: (0, 0)),
                  pl.BlockSpec((1, d), lambda i: (0, 0)),
                  pl.BlockSpec((1, d), lambda i: (0, 0)),
                  pl.BlockSpec((BM, d), lambda i: (i, 0))],
        out_specs=pl.BlockSpec((BM, d), lambda i: (i, 0)),
        out_shape=jax.ShapeDtypeStruct((n, d), jnp.float32),
    )(q, stats, g2, be2, x)

    return h
